# baseline (speedup 1.0000x reference)
import functools

import jax
import jax.numpy as jnp
from jax.experimental import pallas as pl

B = 32
INPUT_DIM = 128
EMBED_DIM = 32
N_ACTIONS = 4
CAPACITY = 250000
P = 50
DELTA = 1e-3

CB = 2048


def _encoder_body(x_ref, w1_ref, b1_ref, w2_ref, b2_ref, q_ref, qsq_ref):
    h = jax.nn.relu(jnp.dot(x_ref[...], w1_ref[...]) + b1_ref[...])
    q = jax.nn.relu(jnp.dot(h, w2_ref[...]) + b2_ref[...])
    q_ref[...] = q
    qsq_ref[...] = jnp.sum(q * q, axis=1, keepdims=True)


def _scores_body(q_ref, qsq_ref, keys_ref, out_ref):
    nb = pl.program_id(1)
    k = keys_ref[0]
    dots = jax.lax.dot_general(
        q_ref[...], k, (((1,), (1,)), ((), ())))
    k_sq = jnp.sum(k * k, axis=1)
    dist = qsq_ref[...] + k_sq[None, :] - 2.0 * dots
    scores = 1.0 / (dist + DELTA)
    col = nb * CB + jax.lax.broadcasted_iota(jnp.int32, scores.shape, 1)
    out_ref[0] = jnp.where(col < CAPACITY, scores, -jnp.inf)


def kernel(x, W1, b1, W2, b2, dnd_keys, dnd_values):
    q, q_sq = pl.pallas_call(
        _encoder_body,
        out_shape=(
            jax.ShapeDtypeStruct((B, EMBED_DIM), jnp.float32),
            jax.ShapeDtypeStruct((B, 1), jnp.float32),
        ),
    )(x, W1, b1.reshape(1, EMBED_DIM), W2, b2.reshape(1, EMBED_DIM))

    nblk = (CAPACITY + CB - 1) // CB
    scores_all = pl.pallas_call(
        _scores_body,
        grid=(N_ACTIONS, nblk),
        in_specs=[
            pl.BlockSpec((B, EMBED_DIM), lambda a, nb: (0, 0)),
            pl.BlockSpec((B, 1), lambda a, nb: (0, 0)),
            pl.BlockSpec((1, CB, EMBED_DIM), lambda a, nb: (a, nb, 0)),
        ],
        out_specs=pl.BlockSpec((1, B, CB), lambda a, nb: (a, 0, nb)),
        out_shape=jax.ShapeDtypeStruct((N_ACTIONS, B, CAPACITY), jnp.float32),
    )(q, q_sq, dnd_keys)

    scores_ab, idx_ab = jax.lax.top_k(scores_all, P)
    scores = scores_ab.transpose(1, 0, 2)
    indexes = idx_ab.transpose(1, 0, 2)
    neigh_v = jnp.take_along_axis(
        dnd_values[:, None, :], idx_ab, axis=2).transpose(1, 0, 2)
    w = scores / jnp.sum(scores, axis=-1, keepdims=True)
    q_vals = jnp.sum(w * neigh_v, axis=-1)
    values = jnp.max(q_vals, axis=-1)
    actions = jnp.argmax(q_vals, axis=-1)
    return (values, actions, indexes, scores, q)


# speedup vs baseline: 21.4173x; 21.4173x over previous
import jax
import jax.numpy as jnp
from jax.experimental import pallas as pl

B = 32
INPUT_DIM = 128
EMBED_DIM = 32
N_ACTIONS = 4
CAPACITY = 250000
P = 50
DELTA = 1e-3

CB = 2048
CHUNK = 128
NBLK = (CAPACITY + CB - 1) // CB
CPAD = NBLK * CB
NCHUNK = CPAD // CHUNK


def _encoder_body(x_ref, w1_ref, b1_ref, w2_ref, b2_ref, q_ref, qsq_ref):
    h = jax.nn.relu(jnp.dot(x_ref[...], w1_ref[...]) + b1_ref[...])
    q = jax.nn.relu(jnp.dot(h, w2_ref[...]) + b2_ref[...])
    q_ref[...] = q
    qsq_ref[...] = jnp.sum(q * q, axis=1, keepdims=True)


def _scores_body(q_ref, qsq_ref, keys_ref, out_ref, cmax_ref):
    nb = pl.program_id(1)
    k = keys_ref[0]
    dots = jax.lax.dot_general(
        q_ref[...], k, (((1,), (1,)), ((), ())))
    k_sq = jnp.sum(k * k, axis=1)
    dist = qsq_ref[...] + k_sq[None, :] - 2.0 * dots
    scores = 1.0 / (dist + DELTA)
    col = nb * CB + jax.lax.broadcasted_iota(jnp.int32, scores.shape, 1)
    scores = jnp.where(col < CAPACITY, scores, -jnp.inf)
    out_ref[0] = scores
    cmax_ref[0, 0] = jnp.max(scores.reshape(B, CB // CHUNK, CHUNK), axis=2)


def kernel(x, W1, b1, W2, b2, dnd_keys, dnd_values):
    q, q_sq = pl.pallas_call(
        _encoder_body,
        out_shape=(
            jax.ShapeDtypeStruct((B, EMBED_DIM), jnp.float32),
            jax.ShapeDtypeStruct((B, 1), jnp.float32),
        ),
    )(x, W1, b1.reshape(1, EMBED_DIM), W2, b2.reshape(1, EMBED_DIM))

    scores_all, cmax = pl.pallas_call(
        _scores_body,
        grid=(N_ACTIONS, NBLK),
        in_specs=[
            pl.BlockSpec((B, EMBED_DIM), lambda a, nb: (0, 0)),
            pl.BlockSpec((B, 1), lambda a, nb: (0, 0)),
            pl.BlockSpec((1, CB, EMBED_DIM), lambda a, nb: (a, nb, 0)),
        ],
        out_specs=(
            pl.BlockSpec((1, B, CB), lambda a, nb: (a, 0, nb)),
            pl.BlockSpec((1, 1, B, CB // CHUNK), lambda a, nb: (a, nb, 0, 0)),
        ),
        out_shape=(
            jax.ShapeDtypeStruct((N_ACTIONS, B, CPAD), jnp.float32),
            jax.ShapeDtypeStruct((N_ACTIONS, NBLK, B, CB // CHUNK), jnp.float32),
        ),
    )(q, q_sq, dnd_keys)
    cmax = cmax.transpose(0, 2, 1, 3).reshape(N_ACTIONS, B, NCHUNK)

    _, chunk_ids = jax.lax.top_k(cmax, P)
    chunk_ids = jnp.sort(chunk_ids, axis=-1)
    scores_4d = scores_all.reshape(N_ACTIONS, B, NCHUNK, CHUNK)
    cand = jnp.take_along_axis(scores_4d, chunk_ids[..., None], axis=2)
    cand = cand.reshape(N_ACTIONS, B, P * CHUNK)
    cscores, cpos = jax.lax.top_k(cand, P)
    chunk_of = jnp.take_along_axis(chunk_ids, cpos // CHUNK, axis=-1)
    idx_ab = chunk_of * CHUNK + cpos % CHUNK

    scores = cscores.transpose(1, 0, 2)
    indexes = idx_ab.transpose(1, 0, 2)
    neigh_v = jnp.take_along_axis(
        dnd_values[:, None, :], idx_ab, axis=2).transpose(1, 0, 2)
    w = scores / jnp.sum(scores, axis=-1, keepdims=True)
    q_vals = jnp.sum(w * neigh_v, axis=-1)
    values = jnp.max(q_vals, axis=-1)
    actions = jnp.argmax(q_vals, axis=-1)
    return (values, actions, indexes, scores, q)


# speedup vs baseline: 30.7750x; 1.4369x over previous
import jax
import jax.numpy as jnp
from jax.experimental import pallas as pl

B = 32
INPUT_DIM = 128
EMBED_DIM = 32
N_ACTIONS = 4
CAPACITY = 250000
P = 50
DELTA = 1e-3

CB = 2048
CHUNK = 128
NBLK = (CAPACITY + CB - 1) // CB
CPAD = NBLK * CB
NCHUNK = CPAD // CHUNK


def _encoder_body(x_ref, w1_ref, b1_ref, w2_ref, b2_ref, q_ref, qsq_ref):
    h = jax.nn.relu(jnp.dot(x_ref[...], w1_ref[...]) + b1_ref[...])
    q = jax.nn.relu(jnp.dot(h, w2_ref[...]) + b2_ref[...])
    q_ref[...] = q
    qsq_ref[...] = jnp.sum(q * q, axis=1, keepdims=True)


def _scores_body(q_ref, qsq_ref, keys_ref, out_ref, cmax_ref):
    nb = pl.program_id(1)
    k = keys_ref[0]
    dots = jax.lax.dot_general(
        q_ref[...], k, (((1,), (1,)), ((), ())))
    k_sq = jnp.sum(k * k, axis=1)
    dist = qsq_ref[...] + k_sq[None, :] - 2.0 * dots
    scores = 1.0 / (dist + DELTA)
    col = nb * CB + jax.lax.broadcasted_iota(jnp.int32, scores.shape, 1)
    scores = jnp.where(col < CAPACITY, scores, -jnp.inf)
    out_ref[0] = scores
    cmax_ref[0, 0] = jnp.max(scores.reshape(B, CB // CHUNK, CHUNK), axis=2)


def _select_body(x_ref, vals_ref, idx_ref):
    rows, n = x_ref.shape
    iota = jax.lax.broadcasted_iota(jnp.int32, (rows, n), 1)
    piota = jax.lax.broadcasted_iota(jnp.int32, (rows, P), 1)

    def step(i, carry):
        x, vals, idx = carry
        m = jnp.max(x, axis=1, keepdims=True)
        im = jnp.min(jnp.where(x == m, iota, n), axis=1, keepdims=True)
        vals = jnp.where(piota == i, m, vals)
        idx = jnp.where(piota == i, im, idx)
        return jnp.where(iota == im, -jnp.inf, x), vals, idx

    _, vals, idx = jax.lax.fori_loop(
        0, P, step,
        (x_ref[...], jnp.zeros((rows, P), jnp.float32),
         jnp.zeros((rows, P), jnp.int32)))
    vals_ref[...] = vals
    idx_ref[...] = idx


def _topk(x):
    rows = x.shape[0]
    return pl.pallas_call(
        _select_body,
        out_shape=(
            jax.ShapeDtypeStruct((rows, P), jnp.float32),
            jax.ShapeDtypeStruct((rows, P), jnp.int32),
        ),
    )(x)


def kernel(x, W1, b1, W2, b2, dnd_keys, dnd_values):
    q, q_sq = pl.pallas_call(
        _encoder_body,
        out_shape=(
            jax.ShapeDtypeStruct((B, EMBED_DIM), jnp.float32),
            jax.ShapeDtypeStruct((B, 1), jnp.float32),
        ),
    )(x, W1, b1.reshape(1, EMBED_DIM), W2, b2.reshape(1, EMBED_DIM))

    scores_all, cmax = pl.pallas_call(
        _scores_body,
        grid=(N_ACTIONS, NBLK),
        in_specs=[
            pl.BlockSpec((B, EMBED_DIM), lambda a, nb: (0, 0)),
            pl.BlockSpec((B, 1), lambda a, nb: (0, 0)),
            pl.BlockSpec((1, CB, EMBED_DIM), lambda a, nb: (a, nb, 0)),
        ],
        out_specs=(
            pl.BlockSpec((1, B, CB), lambda a, nb: (a, 0, nb)),
            pl.BlockSpec((1, 1, B, CB // CHUNK), lambda a, nb: (a, nb, 0, 0)),
        ),
        out_shape=(
            jax.ShapeDtypeStruct((N_ACTIONS, B, CPAD), jnp.float32),
            jax.ShapeDtypeStruct((N_ACTIONS, NBLK, B, CB // CHUNK), jnp.float32),
        ),
    )(q, q_sq, dnd_keys)
    cmax = cmax.transpose(0, 2, 1, 3).reshape(N_ACTIONS, B, NCHUNK)

    _, chunk_ids = _topk(cmax.reshape(N_ACTIONS * B, NCHUNK))
    chunk_ids = jnp.sort(chunk_ids, axis=-1)
    chunk_ids = chunk_ids.reshape(N_ACTIONS, B, P)
    scores_4d = scores_all.reshape(N_ACTIONS, B, NCHUNK, CHUNK)
    cand = jnp.take_along_axis(scores_4d, chunk_ids[..., None], axis=2)
    cscores, cpos = _topk(cand.reshape(N_ACTIONS * B, P * CHUNK))
    cscores = cscores.reshape(N_ACTIONS, B, P)
    cpos = cpos.reshape(N_ACTIONS, B, P)
    chunk_of = jnp.take_along_axis(chunk_ids, cpos // CHUNK, axis=-1)
    idx_ab = chunk_of * CHUNK + cpos % CHUNK

    scores = cscores.transpose(1, 0, 2)
    indexes = idx_ab.transpose(1, 0, 2)
    neigh_v = jnp.take_along_axis(
        dnd_values[:, None, :], idx_ab, axis=2).transpose(1, 0, 2)
    w = scores / jnp.sum(scores, axis=-1, keepdims=True)
    q_vals = jnp.sum(w * neigh_v, axis=-1)
    values = jnp.max(q_vals, axis=-1)
    actions = jnp.argmax(q_vals, axis=-1)
    return (values, actions, indexes, scores, q)
